# Initial kernel scaffold; baseline (speedup 1.0000x reference)
import jax
import jax.numpy as jnp
from jax.experimental import pallas as pl


def kernel(x, edge_indices, Wl1, Wr1, b1, Wl2, Wr2, b2):
    raise NotImplementedError("write your pallas kernel here")


# speedup vs baseline: 9.3280x; 9.3280x over previous
import functools

import jax
import jax.numpy as jnp
from jax import lax
from jax.experimental import pallas as pl
from jax.experimental.pallas import tpu as pltpu
from jax.experimental.pallas import tpu_sc as plsc

_NC = 2
_NS = 16
_NW = _NC * _NS


def _tc1_body(x_ref, wl_ref, y_ref):
    n = x_ref.shape[0]
    y = lax.dot_general(x_ref[...], wl_ref[...], (((1,), (1,)), ((), ())),
                        preferred_element_type=jnp.float32)
    ones = (lax.broadcasted_iota(jnp.int32, (n, 16), 1) == 0).astype(jnp.float32)
    y_ref[...] = jnp.concatenate([y, ones], axis=1)


def _tc2_body(acc_ref, x_ref, wr_ref, b_ref, h_ref):
    n = x_ref.shape[0]
    a = acc_ref[0, :n] + acc_ref[1, :n]
    dh = a.shape[1] - 16
    mean = a[:, :dh] / jnp.maximum(a[:, dh:dh + 1], 1.0)
    pre = mean + b_ref[...] + lax.dot_general(
        x_ref[...], wr_ref[...], (((1,), (1,)), ((), ())),
        preferred_element_type=jnp.float32)
    h_ref[...] = jnp.where(pre >= 0, pre, 0.01 * pre)


def _tc3_body(acc2_ref, acc1_ref, h_ref, wl_ref, wr_ref, b_ref, out_ref):
    n = h_ref.shape[0]
    a1 = acc1_ref[0, :n] + acc1_ref[1, :n]
    dh = a1.shape[1] - 16
    deg = jnp.maximum(a1[:, dh:dh + 1], 1.0)
    mean2 = (acc2_ref[0, :n] + acc2_ref[1, :n]) / deg
    z = (lax.dot_general(mean2, wl_ref[...], (((1,), (1,)), ((), ())),
                         preferred_element_type=jnp.float32)
         + b_ref[...]
         + lax.dot_general(h_ref[...], wr_ref[...], (((1,), (1,)), ((), ())),
                           preferred_element_type=jnp.float32))
    nrm = jnp.sqrt(jnp.sum(z * z, axis=1, keepdims=True))
    out_ref[...] = z / jnp.maximum(nrm, 1e-12)


@functools.lru_cache(maxsize=None)
def _make_segsum(n_nodes, width, n_chunks, blk):
    mesh = plsc.VectorSubcoreMesh(core_axis_name="c", subcore_axis_name="s")
    n_pad = -(-n_nodes // (_NS * 128)) * (_NS * 128)
    rows_sub = n_pad // _NS
    full = rows_sub // 128
    wpr = width // 16

    @functools.partial(
        pl.kernel,
        mesh=mesh,
        out_type=jax.ShapeDtypeStruct((_NC, n_pad, width), jnp.float32),
        scratch_types=[
            pltpu.VMEM((n_chunks, blk), jnp.int32),
            pltpu.VMEM((n_chunks, blk), jnp.int32),
            pltpu.VMEM((blk, width), jnp.float32),
            pltpu.VMEM((128, width), jnp.float32),
            pltpu.VMEM_SHARED((n_pad, width), jnp.float32),
            pltpu.SemaphoreType.DMA,
        ],
        compiler_params=pltpu.CompilerParams(use_tc_tiling_on_sc=False),
    )
    def seg(y_hbm, src_hbm, dst_hbm, out_hbm, src_v, dst_v, rows_v, zbuf, acc_sh, sem):
        c = lax.axis_index("c")
        s = lax.axis_index("s")
        wid = c * _NS + s
        base = s * rows_sub

        zvec = jnp.zeros((16,), jnp.float32)

        def _z(i, carry):
            zbuf[i // wpr, pl.ds((i % wpr) * 16, 16)] = zvec
            return carry

        lax.fori_loop(0, 128 * wpr, _z, 0)
        for j in range(full):
            pltpu.sync_copy(zbuf, acc_sh.at[pl.ds(base + j * 128, 128)])
        plsc.subcore_barrier()

        pltpu.sync_copy(src_hbm.at[wid], src_v)
        pltpu.sync_copy(dst_hbm.at[wid], dst_v)

        def _edge(j, carry):
            pltpu.async_copy(y_hbm.at[src_v.at[j]], rows_v, sem).wait()
            pltpu.sync_copy(rows_v, acc_sh.at[dst_v.at[j]], add=True)
            return carry

        lax.fori_loop(0, n_chunks, _edge, 0)
        plsc.subcore_barrier()

        for j in range(full):
            pltpu.sync_copy(acc_sh.at[pl.ds(base + j * 128, 128)], zbuf)
            pltpu.sync_copy(zbuf, out_hbm.at[c, pl.ds(base + j * 128, 128)])

    return seg


def kernel(x, edge_indices, Wl1, Wr1, b1, Wl2, Wr2, b2):
    n, _ = x.shape
    e = edge_indices.shape[1]
    d_hid = Wl1.shape[0]
    d_out = Wl2.shape[0]
    per_tile = e // _NW
    blk = 80
    n_chunks = per_tile // blk
    w1 = d_hid + 16

    src = edge_indices[0].reshape(_NW, n_chunks, blk)
    dst = edge_indices[1].reshape(_NW, n_chunks, blk)

    y1 = pl.pallas_call(
        _tc1_body, out_shape=jax.ShapeDtypeStruct((n, w1), jnp.float32))(x, Wl1)
    acc1 = _make_segsum(n, w1, n_chunks, blk)(y1, src, dst)
    h = pl.pallas_call(
        _tc2_body, out_shape=jax.ShapeDtypeStruct((n, d_hid), jnp.float32))(
            acc1, x, Wr1, b1.reshape(1, -1))
    acc2 = _make_segsum(n, d_hid, n_chunks, blk)(h, src, dst)
    out = pl.pallas_call(
        _tc3_body, out_shape=jax.ShapeDtypeStruct((n, d_out), jnp.float32))(
            acc2, acc1, h, Wl2, Wr2, b2.reshape(1, -1))
    return out
